# Initial kernel scaffold; baseline (speedup 1.0000x reference)
import jax
import jax.numpy as jnp
from jax.experimental import pallas as pl


def kernel(x, Wg, bg, Wqkv, Wproj, bproj):
    raise NotImplementedError("write your pallas kernel here")


# speedup vs baseline: 1.3119x; 1.3119x over previous
import jax
import jax.numpy as jnp
from jax.experimental import pallas as pl
from jax.experimental.pallas import tpu as pltpu

DIM = 1024
NUM_EXPERTS = 8
NUM_HEADS = 16
TOP_K = 2
DH = DIM // NUM_HEADS
SCALE = DH ** (-0.5)


def _top2_weights(logits):
    T, E = logits.shape
    m = jnp.max(logits, axis=-1, keepdims=True)
    p = jnp.exp(logits - m)
    probs = p / jnp.sum(p, axis=-1, keepdims=True)

    idx = jax.lax.broadcasted_iota(jnp.int32, (T, E), 1)
    big = jnp.int32(E)
    i1 = jnp.min(jnp.where(logits == m, idx, big), axis=-1, keepdims=True)
    mask1 = idx == i1
    logits2 = jnp.where(mask1, -jnp.inf, logits)
    max2 = jnp.max(logits2, axis=-1, keepdims=True)
    i2 = jnp.min(jnp.where(logits2 == max2, idx, big), axis=-1, keepdims=True)
    mask2 = idx == i2
    return jnp.where(mask1 | mask2, probs, 0.0)


def _qkv_body(x_ref, wg_ref, bg_ref, wqkv_ref, qkv_ref, w8_ref):
    e = pl.program_id(0)
    t = pl.program_id(1)
    TB = x_ref.shape[0]
    rows = pl.ds(t * TB, TB)
    x = x_ref[...]

    @pl.when(e == 0)
    def _gate():
        logits = jnp.dot(x, wg_ref[...],
                         preferred_element_type=jnp.float32) + bg_ref[...]
        w8_ref[rows, :] = _top2_weights(logits)

    w8 = w8_ref[rows, :]
    lane = jax.lax.broadcasted_iota(jnp.int32, w8.shape, 1)
    w_e = jnp.sum(jnp.where(lane == e, w8, 0.0), axis=1, keepdims=True)
    contrib = w_e * jnp.dot(x, wqkv_ref[0], preferred_element_type=jnp.float32)

    @pl.when(e == 0)
    def _init():
        qkv_ref[rows, :] = contrib

    @pl.when(e > 0)
    def _acc():
        qkv_ref[rows, :] = qkv_ref[rows, :] + contrib


def _attn_body(qkv_ref, wp_ref, bp_ref, out_ref, att_ref):
    qkv = qkv_ref[...]
    T = qkv.shape[0]
    q = qkv[:, :DIM]
    k = qkv[:, DIM:2 * DIM]
    v = qkv[:, 2 * DIM:]

    r = jax.lax.broadcasted_iota(jnp.int32, (DIM, NUM_HEADS), 0)
    c = jax.lax.broadcasted_iota(jnp.int32, (DIM, NUM_HEADS), 1)
    S = (r // DH == c).astype(jnp.float32)

    for i in range(NUM_HEADS):
        qi = q[:, i * DH:(i + 1) * DH]
        qrep = jnp.concatenate([qi] * NUM_HEADS, axis=1)
        logits = jnp.dot(qrep * k, S,
                         preferred_element_type=jnp.float32) * SCALE
        logits = logits - jnp.max(logits, axis=-1, keepdims=True)
        w = jnp.exp(logits)
        w = w / jnp.sum(w, axis=-1, keepdims=True)
        out_i = jnp.zeros((T, DH), dtype=jnp.float32)
        for j in range(NUM_HEADS):
            out_i = out_i + w[:, j:j + 1] * v[:, j * DH:(j + 1) * DH]
        att_ref[:, i * DH:(i + 1) * DH] = out_i

    out_ref[...] = jnp.dot(att_ref[...], wp_ref[...],
                           preferred_element_type=jnp.float32) + bp_ref[...]


@jax.jit
def kernel(x, Wg, bg, Wqkv, Wproj, bproj):
    B = x.shape[0]
    TB = 256

    qkv = pl.pallas_call(
        _qkv_body,
        grid=(NUM_EXPERTS, B // TB),
        in_specs=[
            pl.BlockSpec((TB, DIM), lambda e, t: (t, 0)),
            pl.BlockSpec((DIM, NUM_EXPERTS), lambda e, t: (0, 0)),
            pl.BlockSpec((1, NUM_EXPERTS), lambda e, t: (0, 0)),
            pl.BlockSpec((1, DIM, 3 * DIM), lambda e, t: (e, 0, 0)),
        ],
        out_specs=pl.BlockSpec((B, 3 * DIM), lambda e, t: (0, 0)),
        out_shape=jax.ShapeDtypeStruct((B, 3 * DIM), jnp.float32),
        scratch_shapes=[pltpu.VMEM((B, NUM_EXPERTS), jnp.float32)],
        compiler_params=pltpu.CompilerParams(
            dimension_semantics=("arbitrary", "arbitrary"),
        ),
    )(x, Wg, bg.reshape(1, NUM_EXPERTS), Wqkv)

    Wp2 = Wproj.reshape(DH, NUM_HEADS, DIM).transpose(1, 0, 2).reshape(DIM, DIM)

    out = pl.pallas_call(
        _attn_body,
        grid=(B // TB,),
        in_specs=[
            pl.BlockSpec((TB, 3 * DIM), lambda t: (t, 0)),
            pl.BlockSpec((DIM, DIM), lambda t: (0, 0)),
            pl.BlockSpec((1, DIM), lambda t: (0, 0)),
        ],
        out_specs=pl.BlockSpec((TB, DIM), lambda t: (t, 0)),
        out_shape=jax.ShapeDtypeStruct((B, DIM), jnp.float32),
        scratch_shapes=[pltpu.VMEM((TB, DIM), jnp.float32)],
        compiler_params=pltpu.CompilerParams(
            dimension_semantics=("parallel",),
        ),
    )(qkv, Wp2, bproj.reshape(1, DIM))

    return out
